# baseline (speedup 1.0000x reference)
import functools

import jax
import jax.numpy as jnp
from jax import lax
from jax.experimental import pallas as pl
from jax.experimental.pallas import tpu as pltpu
from jax.experimental.pallas import tpu_sc as plsc

B = 16384
N_CAT = 26
N_NUM = 8
VOCAB = 100000
CHANNELS = 16
N_COLS = N_CAT + N_NUM
INV = 1.0 / N_COLS

NC = 2
NS = 16
NW = NC * NS
ROWS_PER_W = B // NW
CHUNK = 128
CHUNKS_PER_W = ROWS_PER_W // CHUNK
IDX_W = 32


def _fin_body(part_ref, num_ref, w_ref, b_ref, out_ref):
    b_sum = jnp.sum(b_ref[...], axis=0, keepdims=True)
    out_ref[...] = part_ref[...] + (
        jnp.dot(num_ref[...], w_ref[...], preferred_element_type=jnp.float32)
        + b_sum
    ) * INV


def _finalize(partial, num_feat, lin_w, lin_b):
    return pl.pallas_call(
        _fin_body,
        out_shape=jax.ShapeDtypeStruct((B, CHANNELS), jnp.float32),
    )(partial, num_feat, lin_w, lin_b)


@functools.partial(
    pl.kernel,
    out_type=jax.ShapeDtypeStruct((B, CHANNELS), jnp.float32),
    mesh=plsc.VectorSubcoreMesh(core_axis_name="c", subcore_axis_name="s"),
    compiler_params=pltpu.CompilerParams(use_tc_tiling_on_sc=False),
    scratch_types=[
        pltpu.VMEM((CHUNK, IDX_W), jnp.int32),
        pltpu.VMEM((CHUNK * IDX_W, CHANNELS), jnp.float32),
        pltpu.VMEM((CHUNK, CHANNELS), jnp.float32),
        pltpu.SemaphoreType.DMA,
    ],
)
def _sc_gather(table, idx_p, out, idx_v, rows_v, out_v, sem):
    wid = lax.axis_index("s") * NC + lax.axis_index("c")

    def chunk_body(c, carry):
        row0 = (wid * CHUNKS_PER_W + c) * CHUNK
        pltpu.sync_copy(idx_p.at[pl.ds(row0, CHUNK), pl.ds(0, IDX_W)], idx_v)

        def fire(r, fcarry):
            pltpu.async_copy(
                table.at[idx_v.at[r]],
                rows_v.at[pl.ds(r * IDX_W, IDX_W)],
                sem,
            )
            return fcarry

        lax.fori_loop(0, CHUNK, fire, 0)
        pltpu.make_async_copy(
            table.at[pl.ds(0, CHUNK * IDX_W)], rows_v, sem
        ).wait()

        def row_body(r, rcarry):
            p0 = r * IDX_W
            acc = rows_v[p0]
            for j in range(1, N_CAT):
                acc = acc + rows_v[p0 + j]
            out_v[r] = acc * INV
            return rcarry

        lax.fori_loop(0, CHUNK, row_body, 0)
        pltpu.sync_copy(out_v, out.at[pl.ds(row0, CHUNK)])
        return carry

    lax.fori_loop(0, CHUNKS_PER_W, chunk_body, 0)


def kernel(cat_idx, num_feat, emb_tables, lin_w, lin_b):
    table = emb_tables.reshape(N_CAT * VOCAB, CHANNELS)
    lane = jax.lax.broadcasted_iota(jnp.int32, (B, 128), 1)
    cat_pad = jnp.pad(cat_idx.astype(jnp.int32), ((0, 0), (0, 128 - N_CAT)))
    idx_p = jnp.where(lane < N_CAT, cat_pad + lane * VOCAB, 0)
    partial = _sc_gather(table, idx_p)
    return _finalize(partial, num_feat, lin_w, lin_b)


# speedup vs baseline: 1.4131x; 1.4131x over previous
import functools

import jax
import jax.numpy as jnp
from jax import lax
from jax.experimental import pallas as pl
from jax.experimental.pallas import tpu as pltpu
from jax.experimental.pallas import tpu_sc as plsc

B = 16384
N_CAT = 26
N_NUM = 8
VOCAB = 100000
CHANNELS = 16
N_COLS = N_CAT + N_NUM
INV = 1.0 / N_COLS

NC = 2
NS = 16
NW = NC * NS
ROWS_PER_W = B // NW
CHUNK = 128
CHUNKS_PER_W = ROWS_PER_W // CHUNK
IDX_PER_CHUNK = CHUNK * N_CAT
IDX_PER_W = ROWS_PER_W * N_CAT
IDX_GRP = 128
N_GRP = IDX_PER_CHUNK // IDX_GRP


def _fin_body(part_ref, num_ref, w_ref, b_ref, out_ref):
    b_sum = jnp.sum(b_ref[...], axis=0, keepdims=True)
    out_ref[...] = part_ref[...] + (
        jnp.dot(num_ref[...], w_ref[...], preferred_element_type=jnp.float32)
        + b_sum
    ) * INV


def _finalize(partial, num_feat, lin_w, lin_b):
    return pl.pallas_call(
        _fin_body,
        out_shape=jax.ShapeDtypeStruct((B, CHANNELS), jnp.float32),
    )(partial, num_feat, lin_w, lin_b)


@functools.partial(
    pl.kernel,
    out_type=jax.ShapeDtypeStruct((B, CHANNELS), jnp.float32),
    mesh=plsc.VectorSubcoreMesh(core_axis_name="c", subcore_axis_name="s"),
    compiler_params=pltpu.CompilerParams(use_tc_tiling_on_sc=False),
    scratch_types=[
        pltpu.VMEM((IDX_PER_W,), jnp.int32),
        pltpu.VMEM((IDX_PER_CHUNK, CHANNELS), jnp.float32),
        pltpu.VMEM((CHUNK, CHANNELS), jnp.float32),
        pltpu.SemaphoreType.DMA,
    ],
)
def _sc_gather(table, idx1, out, idx_v, rows_v, out_v, sem):
    wid = lax.axis_index("s") * NC + lax.axis_index("c")
    pltpu.sync_copy(idx1.at[pl.ds(wid * IDX_PER_W, IDX_PER_W)], idx_v)

    def chunk_body(c, carry):
        row0 = (wid * CHUNKS_PER_W + c) * CHUNK

        def fire(g, fcarry):
            pltpu.async_copy(
                table.at[idx_v.at[pl.ds((c * N_GRP + g) * IDX_GRP, IDX_GRP)]],
                rows_v.at[pl.ds(g * IDX_GRP, IDX_GRP)],
                sem,
            )
            return fcarry

        lax.fori_loop(0, N_GRP, fire, 0)
        pltpu.make_async_copy(
            table.at[pl.ds(0, IDX_PER_CHUNK)], rows_v, sem
        ).wait()

        def row_body(r, rcarry):
            p0 = r * N_CAT
            acc = rows_v[p0]
            for j in range(1, N_CAT):
                acc = acc + rows_v[p0 + j]
            out_v[r] = acc * INV
            return rcarry

        lax.fori_loop(0, CHUNK, row_body, 0)
        pltpu.sync_copy(out_v, out.at[pl.ds(row0, CHUNK)])
        return carry

    lax.fori_loop(0, CHUNKS_PER_W, chunk_body, 0)


def kernel(cat_idx, num_feat, emb_tables, lin_w, lin_b):
    table = emb_tables.reshape(N_CAT * VOCAB, CHANNELS)
    offs = (jnp.arange(N_CAT, dtype=jnp.int32) * VOCAB)[None, :]
    idx1 = (cat_idx.astype(jnp.int32) + offs).reshape(B * N_CAT)
    partial = _sc_gather(table, idx1)
    return _finalize(partial, num_feat, lin_w, lin_b)


# speedup vs baseline: 1.4306x; 1.0123x over previous
import functools

import jax
import jax.numpy as jnp
from jax import lax
from jax.experimental import pallas as pl
from jax.experimental.pallas import tpu as pltpu
from jax.experimental.pallas import tpu_sc as plsc

B = 16384
N_CAT = 26
N_NUM = 8
VOCAB = 100000
CHANNELS = 16
N_COLS = N_CAT + N_NUM
INV = 1.0 / N_COLS

NC = 2
NS = 16
NW = NC * NS
ROWS_PER_W = B // NW
CHUNK = 128
CHUNKS_PER_W = ROWS_PER_W // CHUNK
IDX_PER_CHUNK = CHUNK * N_CAT


def _fin_body(part_ref, num_ref, w_ref, b_ref, out_ref):
    b_sum = jnp.sum(b_ref[...], axis=0, keepdims=True)
    out_ref[...] = part_ref[...] + (
        jnp.dot(num_ref[...], w_ref[...], preferred_element_type=jnp.float32)
        + b_sum
    ) * INV


def _finalize(partial, num_feat, lin_w, lin_b):
    return pl.pallas_call(
        _fin_body,
        out_shape=jax.ShapeDtypeStruct((B, CHANNELS), jnp.float32),
    )(partial, num_feat, lin_w, lin_b)


@functools.partial(
    pl.kernel,
    out_type=jax.ShapeDtypeStruct((B, CHANNELS), jnp.float32),
    mesh=plsc.VectorSubcoreMesh(core_axis_name="c", subcore_axis_name="s"),
    compiler_params=pltpu.CompilerParams(use_tc_tiling_on_sc=False),
    scratch_types=[
        pltpu.VMEM((N_CAT, ROWS_PER_W), jnp.int32),
        pltpu.VMEM((IDX_PER_CHUNK, CHANNELS), jnp.float32),
        pltpu.VMEM((CHUNK, CHANNELS), jnp.float32),
        pltpu.SemaphoreType.DMA,
    ],
)
def _sc_gather(table3, idx_t, out, idx_v, rows_v, out_v, sem):
    wid = lax.axis_index("s") * NC + lax.axis_index("c")
    base = wid * ROWS_PER_W
    pltpu.sync_copy(idx_t.at[:, pl.ds(base, ROWS_PER_W)], idx_v)

    def chunk_body(c, carry):
        row0 = base + c * CHUNK
        descs = []
        for j in range(N_CAT):
            descs.append(
                pltpu.async_copy(
                    table3.at[j].at[idx_v.at[j, pl.ds(c * CHUNK, CHUNK)]],
                    rows_v.at[pl.ds(j * CHUNK, CHUNK)],
                    sem,
                )
            )
        for d in descs:
            d.wait()

        def row_body(r, rcarry):
            acc = rows_v[r]
            for j in range(1, N_CAT):
                acc = acc + rows_v[j * CHUNK + r]
            out_v[r] = acc * INV
            return rcarry

        lax.fori_loop(0, CHUNK, row_body, 0)
        pltpu.sync_copy(out_v, out.at[pl.ds(row0, CHUNK)])
        return carry

    lax.fori_loop(0, CHUNKS_PER_W, chunk_body, 0)


def kernel(cat_idx, num_feat, emb_tables, lin_w, lin_b):
    idx_t = jnp.swapaxes(cat_idx.astype(jnp.int32), 0, 1)
    partial = _sc_gather(emb_tables, idx_t)
    return _finalize(partial, num_feat, lin_w, lin_b)
